# Initial kernel scaffold; baseline (speedup 1.0000x reference)
import functools

import jax
import jax.numpy as jnp
from jax import lax
from jax.experimental import pallas as pl
from jax.experimental.pallas import tpu as pltpu
from jax.experimental.pallas import tpu_sc as plsc

NC = 2
NS = 16
NW = NC * NS
D = 64
CHUNK = 128


@functools.lru_cache(maxsize=None)
def _dual_gather(N: int):
    per_w = N // NW
    n_chunks = per_w // CHUNK
    mesh = plsc.VectorSubcoreMesh(core_axis_name="c", subcore_axis_name="s")

    @functools.partial(
        pl.kernel,
        out_type=(
            jax.ShapeDtypeStruct((N, D), jnp.float32),
            jax.ShapeDtypeStruct((N, D), jnp.float32),
        ),
        mesh=mesh,
        scratch_types=[
            pltpu.VMEM((CHUNK,), jnp.int32),
            pltpu.VMEM((CHUNK, D), jnp.float32),
            pltpu.VMEM((CHUNK, D), jnp.float32),
            pltpu.SemaphoreType.DMA,
            pltpu.SemaphoreType.DMA,
        ],
    )
    def k(idx_hbm, amp_hbm, ph_hbm, amp_out, ph_out,
          idx_v, amp_v, ph_v, sem_a, sem_p):
        wid = lax.axis_index("s") * NC + lax.axis_index("c")
        base = wid * per_w

        def body(i, carry):
            off = base + i * CHUNK
            pltpu.sync_copy(idx_hbm.at[pl.ds(off, CHUNK)], idx_v)
            ca = pltpu.async_copy(amp_hbm.at[idx_v], amp_v, sem_a)
            cp = pltpu.async_copy(ph_hbm.at[idx_v], ph_v, sem_p)
            ca.wait()
            cp.wait()
            pltpu.sync_copy(amp_v, amp_out.at[pl.ds(off, CHUNK)])
            pltpu.sync_copy(ph_v, ph_out.at[pl.ds(off, CHUNK)])
            return carry

        lax.fori_loop(0, n_chunks, body, 0)

    return k


def kernel(indices, amplitude_table, phase_table):
    B, L = indices.shape
    N = B * L
    flat = indices.reshape(N)
    amp, ph = _dual_gather(N)(flat, amplitude_table, phase_table)
    return amp.reshape(B, L, D), ph.reshape(B, L, D)


# speedup vs baseline: 4.7058x; 4.7058x over previous
import functools

import jax
import jax.numpy as jnp
from jax import lax
from jax.experimental import pallas as pl
from jax.experimental.pallas import tpu as pltpu
from jax.experimental.pallas import tpu_sc as plsc

NC = 2
NS = 16
NW = NC * NS
D = 64
CHUNK = 128


@functools.lru_cache(maxsize=None)
def _dual_gather(N: int):
    per_w = N // NW
    n_chunks = per_w // CHUNK
    mesh = plsc.VectorSubcoreMesh(core_axis_name="c", subcore_axis_name="s")

    @functools.partial(
        pl.kernel,
        out_type=(
            jax.ShapeDtypeStruct((N, D), jnp.float32),
            jax.ShapeDtypeStruct((N, D), jnp.float32),
        ),
        mesh=mesh,
        scratch_types=[
            pltpu.VMEM((CHUNK,), jnp.int32),
            pltpu.VMEM((CHUNK, D), jnp.float32),
            pltpu.VMEM((CHUNK, D), jnp.float32),
            pltpu.SemaphoreType.DMA,
            pltpu.SemaphoreType.DMA,
        ],
        compiler_params=pltpu.CompilerParams(use_tc_tiling_on_sc=False),
    )
    def k(idx_hbm, amp_hbm, ph_hbm, amp_out, ph_out,
          idx_v, amp_v, ph_v, sem_a, sem_p):
        wid = lax.axis_index("s") * NC + lax.axis_index("c")
        base = wid * per_w

        def body(i, carry):
            off = base + i * CHUNK
            pltpu.sync_copy(idx_hbm.at[pl.ds(off, CHUNK)], idx_v)
            ca = pltpu.async_copy(amp_hbm.at[idx_v], amp_v, sem_a)
            cp = pltpu.async_copy(ph_hbm.at[idx_v], ph_v, sem_p)
            ca.wait()
            cp.wait()
            pltpu.sync_copy(amp_v, amp_out.at[pl.ds(off, CHUNK)])
            pltpu.sync_copy(ph_v, ph_out.at[pl.ds(off, CHUNK)])
            return carry

        lax.fori_loop(0, n_chunks, body, 0)

    return k


def kernel(indices, amplitude_table, phase_table):
    B, L = indices.shape
    N = B * L
    flat = indices.reshape(N)
    amp, ph = _dual_gather(N)(flat, amplitude_table, phase_table)
    return amp.reshape(B, L, D), ph.reshape(B, L, D)


# speedup vs baseline: 5.2994x; 1.1261x over previous
import functools

import jax
import jax.numpy as jnp
from jax import lax
from jax.experimental import pallas as pl
from jax.experimental.pallas import tpu as pltpu
from jax.experimental.pallas import tpu_sc as plsc

NC = 2
NS = 16
NW = NC * NS
D = 64
CHUNK = 128
NBUF = 5
LA = 3


@functools.lru_cache(maxsize=None)
def _dual_gather(N: int):
    per_w = N // NW
    n_chunks = per_w // CHUNK
    n_groups = n_chunks // NBUF
    mesh = plsc.VectorSubcoreMesh(core_axis_name="c", subcore_axis_name="s")

    @functools.partial(
        pl.kernel,
        out_type=(
            jax.ShapeDtypeStruct((N, D), jnp.float32),
            jax.ShapeDtypeStruct((N, D), jnp.float32),
        ),
        mesh=mesh,
        scratch_types=[
            pltpu.VMEM((per_w,), jnp.int32),
            pltpu.VMEM((NBUF, CHUNK, D), jnp.float32),
            pltpu.VMEM((NBUF, CHUNK, D), jnp.float32),
            pltpu.SemaphoreType.DMA((NBUF,)),
            pltpu.SemaphoreType.DMA((NBUF,)),
            pltpu.SemaphoreType.DMA((NBUF,)),
            pltpu.SemaphoreType.DMA((NBUF,)),
        ],
        compiler_params=pltpu.CompilerParams(use_tc_tiling_on_sc=False),
    )
    def k(idx_hbm, amp_hbm, ph_hbm, amp_out, ph_out,
          idx_all, amp_bufs, ph_bufs, ga_sem, gp_sem, oa_sem, op_sem):
        wid = lax.axis_index("s") * NC + lax.axis_index("c")
        base = wid * per_w

        pltpu.sync_copy(idx_hbm.at[pl.ds(base, per_w)], idx_all)

        def gather_desc(j, b):
            idx_sl = idx_all.at[pl.ds(j * CHUNK, CHUNK)]
            return (
                pltpu.make_async_copy(amp_hbm.at[idx_sl], amp_bufs.at[b],
                                      ga_sem.at[b]),
                pltpu.make_async_copy(ph_hbm.at[idx_sl], ph_bufs.at[b],
                                      gp_sem.at[b]),
            )

        def out_desc(j, b):
            sl = pl.ds(base + j * CHUNK, CHUNK)
            return (
                pltpu.make_async_copy(amp_bufs.at[b], amp_out.at[sl],
                                      oa_sem.at[b]),
                pltpu.make_async_copy(ph_bufs.at[b], ph_out.at[sl],
                                      op_sem.at[b]),
            )

        for j in range(LA):
            da, dp = gather_desc(j, j % NBUF)
            da.start()
            dp.start()

        def group(g, carry):
            for b in range(NBUF):
                j = g * NBUF + b
                da, dp = gather_desc(j, b)
                da.wait()
                dp.wait()
                oa, op = out_desc(j, b)
                oa.start()
                op.start()
                bn = (b + LA) % NBUF
                jn = j + LA

                @pl.when(jn >= NBUF)
                def _drain():
                    poa, pop = out_desc(jn - NBUF, bn)
                    poa.wait()
                    pop.wait()

                @pl.when(jn < n_chunks)
                def _prefetch():
                    na, np_ = gather_desc(jn, bn)
                    na.start()
                    np_.start()
            return carry

        lax.fori_loop(0, n_groups, group, 0)

        for j in range(n_chunks - (NBUF - LA), n_chunks):
            oa, op = out_desc(j, j % NBUF)
            oa.wait()
            op.wait()

    return k


def kernel(indices, amplitude_table, phase_table):
    B, L = indices.shape
    N = B * L
    flat = indices.reshape(N)
    amp, ph = _dual_gather(N)(flat, amplitude_table, phase_table)
    return amp.reshape(B, L, D), ph.reshape(B, L, D)


# speedup vs baseline: 5.3064x; 1.0013x over previous
import functools

import jax
import jax.numpy as jnp
from jax import lax
from jax.experimental import pallas as pl
from jax.experimental.pallas import tpu as pltpu
from jax.experimental.pallas import tpu_sc as plsc

NC = 2
NS = 16
NW = NC * NS
CB = 2
NBUF = 4
LA = 2


@functools.lru_cache(maxsize=None)
def _dual_gather(B: int, L: int, D: int):
    bpw = B // NW
    n_chunks = bpw // CB
    n_groups = n_chunks // NBUF
    mesh = plsc.VectorSubcoreMesh(core_axis_name="c", subcore_axis_name="s")

    @functools.partial(
        pl.kernel,
        out_type=(
            jax.ShapeDtypeStruct((B, L, D), jnp.float32),
            jax.ShapeDtypeStruct((B, L, D), jnp.float32),
        ),
        mesh=mesh,
        scratch_types=[
            pltpu.VMEM((bpw, L), jnp.int32),
            pltpu.VMEM((NBUF, CB, L, D), jnp.float32),
            pltpu.VMEM((NBUF, CB, L, D), jnp.float32),
            pltpu.SemaphoreType.DMA((NBUF,)),
            pltpu.SemaphoreType.DMA((NBUF,)),
            pltpu.SemaphoreType.DMA((NBUF,)),
            pltpu.SemaphoreType.DMA((NBUF,)),
        ],
        compiler_params=pltpu.CompilerParams(use_tc_tiling_on_sc=False),
    )
    def k(idx_hbm, amp_hbm, ph_hbm, amp_out, ph_out,
          idx_v, abuf, pbuf, ga_sem, gp_sem, oa_sem, op_sem):
        wid = lax.axis_index("s") * NC + lax.axis_index("c")
        b0 = wid * bpw

        pltpu.sync_copy(idx_hbm.at[pl.ds(b0, bpw), :], idx_v)

        def gather_descs(j, s):
            res = []
            for p in range(CB):
                ib = idx_v.at[CB * j + p]
                res.append(pltpu.make_async_copy(
                    amp_hbm.at[ib], abuf.at[s, p], ga_sem.at[s]))
                res.append(pltpu.make_async_copy(
                    ph_hbm.at[ib], pbuf.at[s, p], gp_sem.at[s]))
            return res

        def out_descs(j, s):
            sl = pl.ds(b0 + CB * j, CB)
            return (
                pltpu.make_async_copy(abuf.at[s], amp_out.at[sl], oa_sem.at[s]),
                pltpu.make_async_copy(pbuf.at[s], ph_out.at[sl], op_sem.at[s]),
            )

        for j in range(LA):
            for d in gather_descs(j, j % NBUF):
                d.start()

        def group(g, carry):
            for s in range(NBUF):
                j = g * NBUF + s
                for d in gather_descs(j, s):
                    d.wait()
                oa, op = out_descs(j, s)
                oa.start()
                op.start()
                sn = (s + LA) % NBUF
                jn = j + LA

                @pl.when(jn >= NBUF)
                def _drain():
                    poa, pop = out_descs(jn - NBUF, sn)
                    poa.wait()
                    pop.wait()

                @pl.when(jn < n_chunks)
                def _prefetch():
                    for d in gather_descs(jn, sn):
                        d.start()
            return carry

        lax.fori_loop(0, n_groups, group, 0)

        for j in range(n_chunks - (NBUF - LA), n_chunks):
            oa, op = out_descs(j, j % NBUF)
            oa.wait()
            op.wait()

    return k


def kernel(indices, amplitude_table, phase_table):
    B, L = indices.shape
    _, D = amplitude_table.shape
    return _dual_gather(B, L, D)(indices, amplitude_table, phase_table)
